# baseline (speedup 1.0000x reference)
import functools

import jax
import jax.numpy as jnp
from jax import lax
from jax.experimental import pallas as pl
from jax.experimental.pallas import tpu as pltpu
from jax.experimental.pallas import tpu_sc as plsc

N = 10000
E = 320000
D = 128

N_PAD = 10240
ROWS_PER_TILE = N_PAD // 16
CHUNK = 128

SCW = 128
SC_CHUNKS = 80
E_PAD = 2 * 16 * SC_CHUNKS * SCW
DEG_CHUNKS = E_PAD // (16 * CHUNK)
DEG_LAG = 4
DEG_W = 16


def _sc_degrees_body(idx_hbm, ones_hbm, zeros_hbm, cnt_out, idx_v, ones_v,
                     cnt_sh, sem):
    c = lax.axis_index("c")
    s = lax.axis_index("s")
    rows = pl.ds(s * ROWS_PER_TILE, ROWS_PER_TILE)
    pltpu.sync_copy(zeros_hbm.at[rows], cnt_sh.at[rows])
    pltpu.sync_copy(ones_hbm, ones_v)
    pltpu.sync_copy(idx_hbm.at[c, s], idx_v)
    plsc.subcore_barrier()

    def body(j, carry):
        pltpu.sync_copy(ones_v, cnt_sh.at[idx_v.at[j]], add=True)
        return carry

    lax.fori_loop(0, DEG_CHUNKS, body, 0)
    plsc.subcore_barrier()
    pltpu.sync_copy(cnt_sh.at[rows], cnt_out.at[c, rows])


def _sc_scatter_body(z_hbm, src_hbm, dst_hbm, zeros_hbm, acc_out, src_v,
                     dst_v, buf, acc_sh, sem):
    c = lax.axis_index("c")
    s = lax.axis_index("s")
    rows = pl.ds(s * ROWS_PER_TILE, ROWS_PER_TILE)
    pltpu.sync_copy(zeros_hbm.at[rows], acc_sh.at[rows])
    pltpu.sync_copy(src_hbm.at[c, s], src_v)
    pltpu.sync_copy(dst_hbm.at[c, s], dst_v)
    plsc.subcore_barrier()

    def body(j, carry):
        pltpu.async_copy(z_hbm.at[src_v.at[j]], buf, sem).wait()
        pltpu.sync_copy(buf, acc_sh.at[dst_v.at[j]], add=True)
        return carry

    lax.fori_loop(0, SC_CHUNKS, body, 0)
    plsc.subcore_barrier()
    pltpu.sync_copy(acc_sh.at[rows], acc_out.at[c, rows])


@functools.cache
def _sc_kernels():
    mesh = plsc.VectorSubcoreMesh(core_axis_name="c", subcore_axis_name="s")
    degrees = pl.kernel(
        _sc_degrees_body,
        out_type=jax.ShapeDtypeStruct((2, N_PAD, D), jnp.float32),
        mesh=mesh,
        scratch_types=[
            pltpu.VMEM((DEG_CHUNKS, CHUNK), jnp.int32),
            pltpu.VMEM((CHUNK, D), jnp.float32),
            pltpu.VMEM_SHARED((N_PAD, D), jnp.float32),
            pltpu.SemaphoreType.DMA,
        ],
    )
    scatter = pl.kernel(
        _sc_scatter_body,
        out_type=jax.ShapeDtypeStruct((2, N_PAD, D), jnp.float32),
        mesh=mesh,
        scratch_types=[
            pltpu.VMEM((SC_CHUNKS, SCW), jnp.int32),
            pltpu.VMEM((SC_CHUNKS, SCW), jnp.int32),
            pltpu.VMEM((SCW, D), jnp.float32),
            pltpu.VMEM_SHARED((N_PAD, D), jnp.float32),
            pltpu.SemaphoreType.DMA,
        ],
    )
    return degrees, scatter


_BN = 1024
_GRID = N_PAD // _BN


def _prep_body(cnt_ref, so_ref, si_ref):
    i = pl.program_id(0)
    row = i * _BN + lax.broadcasted_iota(jnp.int32, (_BN, 1), 0)
    valid = row < N

    def scale(cvec):
        return jnp.where(valid, lax.rsqrt(jnp.maximum(cvec, 1.0)), 0.0)

    so_ref[...] = scale(cnt_ref[0, :, 0:1])
    si_ref[...] = scale(cnt_ref[1, :, 0:1])


def _tc_prep(cnt):
    return pl.pallas_call(
        _prep_body,
        grid=(_GRID,),
        in_specs=[pl.BlockSpec((2, _BN, D), lambda i: (0, i, 0))],
        out_specs=[pl.BlockSpec((_BN, 1), lambda i: (i, 0)),
                   pl.BlockSpec((_BN, 1), lambda i: (i, 0))],
        out_shape=[jax.ShapeDtypeStruct((N_PAD, 1), jnp.float32),
                   jax.ShapeDtypeStruct((N_PAD, 1), jnp.float32)],
    )(cnt)


def _stage0_body(x_ref, so_ref, w_ref, z_ref):
    z_ref[...] = jnp.dot(x_ref[...] * so_ref[...], w_ref[...],
                         preferred_element_type=jnp.float32)


def _tc_stage0(x, s_out, W):
    return pl.pallas_call(
        _stage0_body,
        grid=(_GRID,),
        in_specs=[pl.BlockSpec((_BN, D), lambda i: (i, 0)),
                  pl.BlockSpec((_BN, 1), lambda i: (i, 0)),
                  pl.BlockSpec((D, D), lambda i: (0, 0))],
        out_specs=pl.BlockSpec((_BN, D), lambda i: (i, 0)),
        out_shape=jax.ShapeDtypeStruct((N_PAD, D), jnp.float32),
    )(x, s_out, W)


def _stage_body(a_ref, si_ref, so_ref, b_ref, w_ref, z_ref):
    a = a_ref[0] + a_ref[1]
    h = jnp.maximum(a * si_ref[...] + b_ref[...], 0.0)
    z_ref[...] = jnp.dot(h * so_ref[...], w_ref[...],
                         preferred_element_type=jnp.float32)


def _tc_stage(a, s_in, s_out, b, W):
    return pl.pallas_call(
        _stage_body,
        grid=(_GRID,),
        in_specs=[pl.BlockSpec((2, _BN, D), lambda i: (0, i, 0)),
                  pl.BlockSpec((_BN, 1), lambda i: (i, 0)),
                  pl.BlockSpec((_BN, 1), lambda i: (i, 0)),
                  pl.BlockSpec((1, D), lambda i: (0, 0)),
                  pl.BlockSpec((D, D), lambda i: (0, 0))],
        out_specs=pl.BlockSpec((_BN, D), lambda i: (i, 0)),
        out_shape=jax.ShapeDtypeStruct((N_PAD, D), jnp.float32),
    )(a, s_in, s_out, b, W)


def _final_body(a_ref, si_ref, b2_ref, wo_ref, bo_ref, out_ref, acc_ref):
    i = pl.program_id(0)

    @pl.when(i == 0)
    def _():
        acc_ref[...] = jnp.zeros_like(acc_ref)

    a = a_ref[0] + a_ref[1]
    acc_ref[0:1, :] += jnp.sum(a * si_ref[...], axis=0, keepdims=True)

    @pl.when(i == _GRID - 1)
    def _():
        pooled = acc_ref[0:1, :] + jnp.float32(N) * b2_ref[...]
        out_ref[...] = jnp.dot(pooled, wo_ref[...],
                               preferred_element_type=jnp.float32) + bo_ref[...]


def _tc_final(a, s_in, b2, W_out, b_out):
    return pl.pallas_call(
        _final_body,
        grid=(_GRID,),
        in_specs=[pl.BlockSpec((2, _BN, D), lambda i: (0, i, 0)),
                  pl.BlockSpec((_BN, 1), lambda i: (i, 0)),
                  pl.BlockSpec((1, D), lambda i: (0, 0)),
                  pl.BlockSpec((D, D), lambda i: (0, 0)),
                  pl.BlockSpec((1, D), lambda i: (0, 0))],
        out_specs=pl.BlockSpec((1, D), lambda i: (0, 0)),
        out_shape=jax.ShapeDtypeStruct((1, D), jnp.float32),
        scratch_shapes=[pltpu.VMEM((8, D), jnp.float32)],
    )(a, s_in, b2, W_out, b_out)


@jax.jit
def _run(x, W0, b0, W1, b1, W2, b2, W_out, b_out, edge_index):
    f32 = jnp.float32
    x_pad = jnp.zeros((N_PAD, D), f32).at[:N].set(x)

    pad_idx = jnp.full((E_PAD - E,), N, jnp.int32)
    src = jnp.concatenate([edge_index[0], pad_idx])
    dst = jnp.concatenate([edge_index[1], pad_idx])
    deg_idx = jnp.stack([src, dst]).reshape(2, 16, DEG_CHUNKS, CHUNK)
    src_r = src.reshape(2, 16, SC_CHUNKS, SCW)
    dst_r = dst.reshape(2, 16, SC_CHUNKS, SCW)

    ones_deg = jnp.ones((CHUNK, D), f32)
    zeros128 = jnp.zeros((N_PAD, D), f32)

    sc_degrees, sc_scatter = _sc_kernels()
    cnt = sc_degrees(deg_idx, ones_deg, zeros128)
    s_out, s_in = _tc_prep(cnt)

    z = _tc_stage0(x_pad, s_out, W0)
    a = sc_scatter(z, src_r, dst_r, zeros128)
    z = _tc_stage(a, s_in, s_out, b0.reshape(1, D), W1)
    a = sc_scatter(z, src_r, dst_r, zeros128)
    z = _tc_stage(a, s_in, s_out, b1.reshape(1, D), W2)
    a = sc_scatter(z, src_r, dst_r, zeros128)
    return _tc_final(a, s_in, b2.reshape(1, D), W_out, b_out.reshape(1, D))


def kernel(x, W0, b0, W1, b1, W2, b2, W_out, b_out, edge_index):
    return _run(x, W0, b0, W1, b1, W2, b2, W_out, b_out, edge_index)


# speedup vs baseline: 2.7354x; 2.7354x over previous
import functools

import jax
import jax.numpy as jnp
from jax import lax
from jax.experimental import pallas as pl
from jax.experimental.pallas import tpu as pltpu
from jax.experimental.pallas import tpu_sc as plsc

N = 10000
E = 320000
D = 128

N_PAD = 10240
ROWS_PER_TILE = N_PAD // 16
CHUNK = 128

SCW = 128
SC_CHUNKS = 80
E_PAD = 2 * 16 * SC_CHUNKS * SCW
DEG_CHUNKS = E_PAD // (16 * CHUNK)
DEG_LAG = 4
DEG_W = 16


def _sc_degrees_body(idx_hbm, ones_hbm, zeros_hbm, cnt_out, idx_v, ones_v,
                     cnt_sh, sem):
    c = lax.axis_index("c")
    s = lax.axis_index("s")
    rows = pl.ds(s * ROWS_PER_TILE, ROWS_PER_TILE)
    pltpu.sync_copy(zeros_hbm.at[rows], cnt_sh.at[rows])
    pltpu.sync_copy(ones_hbm, ones_v)
    pltpu.sync_copy(idx_hbm.at[c, s], idx_v)
    plsc.subcore_barrier()

    def body(j, carry):
        pltpu.sync_copy(ones_v, cnt_sh.at[idx_v.at[j]], add=True)
        return carry

    lax.fori_loop(0, DEG_CHUNKS, body, 0)
    plsc.subcore_barrier()
    pltpu.sync_copy(cnt_sh.at[rows], cnt_out.at[c, rows])


def _sc_scatter_body(z_hbm, src_hbm, dst_hbm, zeros_hbm, acc_out, src_v,
                     dst_v, buf, acc_sh, sem):
    c = lax.axis_index("c")
    s = lax.axis_index("s")
    rows = pl.ds(s * ROWS_PER_TILE, ROWS_PER_TILE)
    pltpu.sync_copy(zeros_hbm.at[rows], acc_sh.at[rows])
    pltpu.sync_copy(src_hbm.at[c, s], src_v)
    pltpu.sync_copy(dst_hbm.at[c, s], dst_v)
    plsc.subcore_barrier()

    def body(j, carry):
        pltpu.async_copy(z_hbm.at[src_v.at[j]], buf, sem).wait()
        pltpu.sync_copy(buf, acc_sh.at[dst_v.at[j]], add=True)
        return carry

    lax.fori_loop(0, SC_CHUNKS, body, 0)
    plsc.subcore_barrier()
    pltpu.sync_copy(acc_sh.at[rows], acc_out.at[c, rows])


@functools.cache
def _sc_kernels():
    mesh = plsc.VectorSubcoreMesh(core_axis_name="c", subcore_axis_name="s")
    degrees = pl.kernel(
        _sc_degrees_body,
        out_type=jax.ShapeDtypeStruct((2, N_PAD, D), jnp.float32),
        mesh=mesh,
        scratch_types=[
            pltpu.VMEM((DEG_CHUNKS, CHUNK), jnp.int32),
            pltpu.VMEM((CHUNK, D), jnp.float32),
            pltpu.VMEM_SHARED((N_PAD, D), jnp.float32),
            pltpu.SemaphoreType.DMA,
        ],
    )
    scatter = pl.kernel(
        _sc_scatter_body,
        out_type=jax.ShapeDtypeStruct((2, N_PAD, D), jnp.float32),
        mesh=mesh,
        scratch_types=[
            pltpu.VMEM((SC_CHUNKS, SCW), jnp.int32),
            pltpu.VMEM((SC_CHUNKS, SCW), jnp.int32),
            pltpu.VMEM((SCW, D), jnp.float32),
            pltpu.VMEM_SHARED((N_PAD, D), jnp.float32),
            pltpu.SemaphoreType.DMA,
        ],
    )
    return degrees, scatter


_BN = 1024
_GRID = N_PAD // _BN


def _prep_body(cnt_ref, so_ref, si_ref):
    i = pl.program_id(0)
    row = i * _BN + lax.broadcasted_iota(jnp.int32, (_BN, 1), 0)
    valid = row < N

    def scale(cvec):
        return jnp.where(valid, lax.rsqrt(jnp.maximum(cvec, 1.0)), 0.0)

    so_ref[...] = scale(cnt_ref[0, :, 0:1])
    si_ref[...] = scale(cnt_ref[1, :, 0:1])


def _tc_prep(cnt):
    return pl.pallas_call(
        _prep_body,
        grid=(_GRID,),
        in_specs=[pl.BlockSpec((2, _BN, D), lambda i: (0, i, 0))],
        out_specs=[pl.BlockSpec((_BN, 1), lambda i: (i, 0)),
                   pl.BlockSpec((_BN, 1), lambda i: (i, 0))],
        out_shape=[jax.ShapeDtypeStruct((N_PAD, 1), jnp.float32),
                   jax.ShapeDtypeStruct((N_PAD, 1), jnp.float32)],
    )(cnt)


def _stage0_body(x_ref, so_ref, w_ref, z_ref):
    z_ref[...] = jnp.dot(x_ref[...] * so_ref[...], w_ref[...],
                         preferred_element_type=jnp.float32)


def _tc_stage0(x, s_out, W):
    return pl.pallas_call(
        _stage0_body,
        grid=(_GRID,),
        in_specs=[pl.BlockSpec((_BN, D), lambda i: (i, 0)),
                  pl.BlockSpec((_BN, 1), lambda i: (i, 0)),
                  pl.BlockSpec((D, D), lambda i: (0, 0))],
        out_specs=pl.BlockSpec((_BN, D), lambda i: (i, 0)),
        out_shape=jax.ShapeDtypeStruct((N_PAD, D), jnp.float32),
    )(x, s_out, W)


def _stage_body(a_ref, si_ref, so_ref, b_ref, w_ref, z_ref):
    a = a_ref[0] + a_ref[1]
    h = jnp.maximum(a * si_ref[...] + b_ref[...], 0.0)
    z_ref[...] = jnp.dot(h * so_ref[...], w_ref[...],
                         preferred_element_type=jnp.float32)


def _tc_stage(a, s_in, s_out, b, W):
    return pl.pallas_call(
        _stage_body,
        grid=(_GRID,),
        in_specs=[pl.BlockSpec((2, _BN, D), lambda i: (0, i, 0)),
                  pl.BlockSpec((_BN, 1), lambda i: (i, 0)),
                  pl.BlockSpec((_BN, 1), lambda i: (i, 0)),
                  pl.BlockSpec((1, D), lambda i: (0, 0)),
                  pl.BlockSpec((D, D), lambda i: (0, 0))],
        out_specs=pl.BlockSpec((_BN, D), lambda i: (i, 0)),
        out_shape=jax.ShapeDtypeStruct((N_PAD, D), jnp.float32),
    )(a, s_in, s_out, b, W)


def _final_body(a_ref, si_ref, b2_ref, wo_ref, bo_ref, out_ref, acc_ref):
    i = pl.program_id(0)

    @pl.when(i == 0)
    def _():
        acc_ref[...] = jnp.zeros_like(acc_ref)

    a = a_ref[0] + a_ref[1]
    acc_ref[0:1, :] += jnp.sum(a * si_ref[...], axis=0, keepdims=True)

    @pl.when(i == _GRID - 1)
    def _():
        pooled = acc_ref[0:1, :] + jnp.float32(N) * b2_ref[...]
        out_ref[...] = jnp.dot(pooled, wo_ref[...],
                               preferred_element_type=jnp.float32) + bo_ref[...]


def _tc_final(a, s_in, b2, W_out, b_out):
    return pl.pallas_call(
        _final_body,
        grid=(_GRID,),
        in_specs=[pl.BlockSpec((2, _BN, D), lambda i: (0, i, 0)),
                  pl.BlockSpec((_BN, 1), lambda i: (i, 0)),
                  pl.BlockSpec((1, D), lambda i: (0, 0)),
                  pl.BlockSpec((D, D), lambda i: (0, 0)),
                  pl.BlockSpec((1, D), lambda i: (0, 0))],
        out_specs=pl.BlockSpec((1, D), lambda i: (0, 0)),
        out_shape=jax.ShapeDtypeStruct((1, D), jnp.float32),
        scratch_shapes=[pltpu.VMEM((8, D), jnp.float32)],
    )(a, s_in, b2, W_out, b_out)


@jax.jit
def _run(x, W0, b0, W1, b1, W2, b2, W_out, b_out, edge_index):
    f32 = jnp.float32
    x_pad = jnp.zeros((N_PAD, D), f32).at[:N].set(x)

    pad_idx = N + jnp.arange(E_PAD - E, dtype=jnp.int32) % (N_PAD - N)
    src = jnp.concatenate([edge_index[0], pad_idx])
    dst = jnp.concatenate([edge_index[1], pad_idx])
    deg_idx = jnp.stack([src, dst]).reshape(2, 16, DEG_CHUNKS, CHUNK)
    src_r = src.reshape(2, 16, SC_CHUNKS, SCW)
    dst_r = dst.reshape(2, 16, SC_CHUNKS, SCW)

    ones_deg = jnp.ones((CHUNK, D), f32)
    zeros128 = jnp.zeros((N_PAD, D), f32)

    sc_degrees, sc_scatter = _sc_kernels()
    cnt = sc_degrees(deg_idx, ones_deg, zeros128)
    s_out, s_in = _tc_prep(cnt)

    z = _tc_stage0(x_pad, s_out, W0)
    a = sc_scatter(z, src_r, dst_r, zeros128)
    z = _tc_stage(a, s_in, s_out, b0.reshape(1, D), W1)
    a = sc_scatter(z, src_r, dst_r, zeros128)
    z = _tc_stage(a, s_in, s_out, b1.reshape(1, D), W2)
    a = sc_scatter(z, src_r, dst_r, zeros128)
    return _tc_final(a, s_in, b2.reshape(1, D), W_out, b_out.reshape(1, D))


def kernel(x, W0, b0, W1, b1, W2, b2, W_out, b_out, edge_index):
    return _run(x, W0, b0, W1, b1, W2, b2, W_out, b_out, edge_index)
